# baseline (speedup 1.0000x reference)
import functools

import jax
import jax.numpy as jnp
from jax import lax
from jax.experimental import pallas as pl
from jax.experimental.pallas import tpu as pltpu
from jax.experimental.pallas import tpu_sc as plsc

M = 64
S = 12
R = 3
D = 1024
T = M * R * S
B = 4

NC, NS, L = 2, 16, 16
NW = NC * NS
TPW = (B * T) // NW
TRI_PW = TPW // (R * S)
CH = 8
NCH = TPW // CH
DC = D // L


def _sc_body(x_hbm, wt_hbm, wr_hbm, wk_hbm, lat_hbm, cp_hbm,
             wr_v, wk_v, wt_v, p36_v, xb_v, lb_v):
    cid = lax.axis_index("c")
    sid = lax.axis_index("s")
    wid = sid * NC + cid
    b = wid // 8
    t0 = (wid % 8) * TPW

    pltpu.sync_copy(wr_hbm, wr_v)
    pltpu.sync_copy(wk_hbm, wk_v)
    pltpu.sync_copy(wt_hbm.at[pl.ds((wid % 8) * TRI_PW, TRI_PW)], wt_v)

    def build_row(r, carry):
        role = r // S
        tok = r % S

        def col(cc, carry2):
            sl = pl.ds(cc * L, L)
            p36_v[r, sl] = wr_v[role, sl] + wk_v[tok, sl]
            return carry2

        return lax.fori_loop(0, DC, col, carry)

    lax.fori_loop(0, 36, build_row, 0)

    def chunk(j, carry):
        row0 = j * CH
        pltpu.sync_copy(x_hbm.at[b, pl.ds(t0 + row0, CH)], xb_v)
        pltpu.sync_copy(xb_v, cp_hbm.at[b, pl.ds(t0 + row0, CH)])

        def rowf(l, carry2):
            g = row0 + l
            p36r = g % 36
            wtr = g // 36

            def col(cc, carry3):
                sl = pl.ds(cc * L, L)
                lb_v[l, sl] = xb_v[l, sl] + p36_v[p36r, sl] + wt_v[wtr, sl]
                return carry3

            return lax.fori_loop(0, DC, col, carry2)

        lax.fori_loop(0, CH, rowf, 0)
        pltpu.sync_copy(lb_v, lat_hbm.at[b, pl.ds(t0 + row0, CH)])
        return carry

    lax.fori_loop(0, NCH, chunk, 0)


def kernel(token_embeds, pad_mask, W_triple, W_role, W_tokpos):
    out_sds = jax.ShapeDtypeStruct((B, T, D), token_embeds.dtype)
    f32 = jnp.float32
    run = functools.partial(
        pl.kernel,
        out_type=[out_sds, out_sds],
        mesh=plsc.VectorSubcoreMesh(core_axis_name="c", subcore_axis_name="s"),
        scratch_types=[
            pltpu.VMEM((R, D), f32),
            pltpu.VMEM((S, D), f32),
            pltpu.VMEM((TRI_PW, D), f32),
            pltpu.VMEM((36, D), f32),
            pltpu.VMEM((CH, D), f32),
            pltpu.VMEM((CH, D), f32),
        ],
    )(_sc_body)
    latent, copy = run(token_embeds, W_triple, W_role, W_tokpos)
    return (latent, copy)
